# Initial kernel scaffold; baseline (speedup 1.0000x reference)
import jax
import jax.numpy as jnp
from jax.experimental import pallas as pl


def kernel(x, embedding_matrix):
    raise NotImplementedError("write your pallas kernel here")


# speedup vs baseline: 1.8770x; 1.8770x over previous
import functools

import jax
import jax.numpy as jnp
from jax import lax
from jax.experimental import pallas as pl
from jax.experimental.pallas import tpu as pltpu
from jax.experimental.pallas import tpu_sc as plsc

_NW = 32
_G = 128
_NBUF = 8


def _gather_rows(idx3, table, n_g, d):
    n_rows = _NW * n_g * _G
    mesh = plsc.VectorSubcoreMesh(core_axis_name="c", subcore_axis_name="s")

    @functools.partial(
        pl.kernel,
        mesh=mesh,
        out_type=jax.ShapeDtypeStruct((n_rows, d), jnp.float32),
        scratch_types=(
            [pltpu.VMEM((n_g, _G), jnp.int32),
             pltpu.VMEM((_NBUF, _G, d), jnp.float32)]
            + [pltpu.SemaphoreType.DMA] * (2 * _NBUF)
        ),
        compiler_params=pltpu.CompilerParams(use_tc_tiling_on_sc=False),
    )
    def body(idx_hbm, table_hbm, out_hbm, idx_v, rows_v, *sems):
        gsem, osem = sems[:_NBUF], sems[_NBUF:]
        wid = lax.axis_index("s") * 2 + lax.axis_index("c")
        base = wid * (n_g * _G)
        pltpu.sync_copy(idx_hbm.at[wid], idx_v)

        def fire(g, b):
            pltpu.async_copy(table_hbm.at[idx_v.at[g]], rows_v.at[b], gsem[b])

        def wait_gather(g, b):
            pltpu.make_async_copy(
                table_hbm.at[idx_v.at[g]], rows_v.at[b], gsem[b]).wait()

        def store(g, b):
            pltpu.async_copy(
                rows_v.at[b], out_hbm.at[pl.ds(base + g * _G, _G)], osem[b])

        def wait_store(g, b):
            pltpu.make_async_copy(
                rows_v.at[b], out_hbm.at[pl.ds(base + g * _G, _G)],
                osem[b]).wait()

        for b in range(_NBUF):
            fire(b, b)

        n_outer = n_g // _NBUF

        def outer(o, carry):
            for b in range(_NBUF):
                g = o * _NBUF + b
                wait_gather(g, b)
                store(g, b)
                wait_store(g, b)
                fire(g + _NBUF, b)
            return carry

        lax.fori_loop(0, n_outer - 1, outer, 0)

        for b in range(_NBUF):
            g = (n_outer - 1) * _NBUF + b
            wait_gather(g, b)
            store(g, b)
        for b in range(_NBUF):
            wait_store((n_outer - 1) * _NBUF + b, b)

    return body(idx3, table)


def kernel(x, embedding_matrix):
    b, h = x.shape
    v, d = embedding_matrix.shape
    n = b * h
    n_g = n // (_NW * _G)
    idx3 = x.reshape(_NW, n_g, _G).astype(jnp.int32)
    out = _gather_rows(idx3, embedding_matrix, n_g, d)
    return out.reshape(b, h, d)
